# baseline (speedup 1.0000x reference)
import functools

import jax
import jax.numpy as jnp
from jax import lax
from jax.experimental import pallas as pl
from jax.experimental.pallas import tpu as pltpu
from jax.experimental.pallas import tpu_sc as plsc

B = 16384
ED = 128
NF = 128

NC = 2
NS = 16
NW = NC * NS
CHUNK = 128
N_CHUNK = B // NW // CHUNK
N_IDX_ROWS = B // CHUNK


def _sc_gather_body(uid_hbm, iid_hbm, utab_hbm, itab_hbm,
                    uout_hbm, iout_hbm, idx_v, rows_v, sem):
    wid = lax.axis_index("s") * NC + lax.axis_index("c")
    r0 = wid * N_CHUNK

    pltpu.sync_copy(uid_hbm.at[pl.ds(r0, N_CHUNK)], idx_v)
    cps = [pltpu.async_copy(utab_hbm.at[idx_v.at[j]], rows_v.at[j], sem)
           for j in range(N_CHUNK)]
    for cp in cps:
        cp.wait()
    pltpu.sync_copy(rows_v, uout_hbm.at[pl.ds(r0, N_CHUNK)])

    pltpu.sync_copy(iid_hbm.at[pl.ds(r0, N_CHUNK)], idx_v)
    cps = [pltpu.async_copy(itab_hbm.at[idx_v.at[j]], rows_v.at[j], sem)
           for j in range(N_CHUNK)]
    for cp in cps:
        cp.wait()
    pltpu.sync_copy(rows_v, iout_hbm.at[pl.ds(r0, N_CHUNK)])


def _sc_gather(user_ids2d, item_ids2d, user_table, item_table):
    mesh = plsc.VectorSubcoreMesh(core_axis_name="c", subcore_axis_name="s",
                                  num_cores=NC, num_subcores=NS)
    out_t = jax.ShapeDtypeStruct((N_IDX_ROWS, CHUNK, ED), jnp.float32)
    f = pl.kernel(
        _sc_gather_body,
        out_type=(out_t, out_t),
        mesh=mesh,
        scratch_types=[
            pltpu.VMEM((N_CHUNK, CHUNK), jnp.int32),
            pltpu.VMEM((N_CHUNK, CHUNK, ED), jnp.float32),
            pltpu.SemaphoreType.DMA,
        ],
    )
    return f(user_ids2d, item_ids2d, user_table, item_table)


def _content_body(x_ref, w1_ref, b1_ref, w2_ref, b2_ref, c_ref):
    bf = jnp.bfloat16
    f32 = jnp.float32
    x = x_ref[...].astype(bf)
    h = jnp.maximum(
        jnp.dot(x, w1_ref[...].astype(bf),
                preferred_element_type=f32) + b1_ref[...], 0.0)
    c = jnp.dot(h.astype(bf), w2_ref[...].astype(bf),
                preferred_element_type=f32) + b2_ref[...]
    c_ref[...] = c.astype(bf)


def _content(x, W1, b1, W2, b2, bs=2048):
    nblk = B // bs
    row_blk = lambda idx: (idx, 0)
    whole = lambda idx: (0, 0)
    return pl.pallas_call(
        _content_body,
        grid=(nblk,),
        in_specs=[
            pl.BlockSpec((bs, NF), row_blk),
            pl.BlockSpec((NF, ED), whole),
            pl.BlockSpec((1, ED), whole),
            pl.BlockSpec((ED, ED), whole),
            pl.BlockSpec((1, ED), whole),
        ],
        out_specs=pl.BlockSpec((bs, ED), row_blk),
        out_shape=jax.ShapeDtypeStruct((B, ED), jnp.bfloat16),
    )(x, W1, b1.reshape(1, ED), W2, b2.reshape(1, ED))


def _combine_body(u_ref, i_ref, c_ref, w3_ref, b3_ref, w4_ref, b4_ref,
                  o_ref):
    bf = jnp.bfloat16
    f32 = jnp.float32
    acc = (jnp.dot(u_ref[...].astype(bf), w3_ref[0:ED, :].astype(bf),
                   preferred_element_type=f32)
           + jnp.dot(i_ref[...].astype(bf), w3_ref[ED:2 * ED, :].astype(bf),
                     preferred_element_type=f32)
           + jnp.dot(c_ref[...], w3_ref[2 * ED:3 * ED, :].astype(bf),
                     preferred_element_type=f32)
           + b3_ref[...])
    p = jnp.maximum(acc, 0.0)
    z = jnp.dot(p.astype(bf), w4_ref[...].astype(bf),
                preferred_element_type=f32) + b4_ref[...]
    o_ref[...] = jax.nn.sigmoid(z)


def _combine(u, i, c, W3, b3, W4, b4, bs=2048):
    nblk = B // bs
    row_blk = lambda idx: (idx, 0)
    whole = lambda idx: (0, 0)
    return pl.pallas_call(
        _combine_body,
        grid=(nblk,),
        in_specs=[
            pl.BlockSpec((bs, ED), row_blk),
            pl.BlockSpec((bs, ED), row_blk),
            pl.BlockSpec((bs, ED), row_blk),
            pl.BlockSpec((3 * ED, ED), whole),
            pl.BlockSpec((1, ED), whole),
            pl.BlockSpec((ED, 1), whole),
            pl.BlockSpec((1, 1), whole),
        ],
        out_specs=pl.BlockSpec((bs, 1), row_blk),
        out_shape=jax.ShapeDtypeStruct((B, 1), jnp.float32),
    )(u, i, c, W3, b3.reshape(1, ED), W4, b4.reshape(1, 1))


def kernel(user_ids, item_ids, item_features, user_table, item_table,
           W1, b1, W2, b2, W3, b3, W4, b4):
    uid2 = user_ids.astype(jnp.int32).reshape(N_IDX_ROWS, CHUNK)
    iid2 = item_ids.astype(jnp.int32).reshape(N_IDX_ROWS, CHUNK)
    u3, i3 = _sc_gather(uid2, iid2, user_table, item_table)
    c = _content(item_features, W1, b1, W2, b2)
    u = u3.reshape(B, ED)
    i = i3.reshape(B, ED)
    return _combine(u, i, c, W3, b3, W4, b4)


# speedup vs baseline: 1.1453x; 1.1453x over previous
import functools

import jax
import jax.numpy as jnp
from jax import lax
from jax.experimental import pallas as pl
from jax.experimental.pallas import tpu as pltpu
from jax.experimental.pallas import tpu_sc as plsc

B = 16384
ED = 128
NF = 128

NC = 2
NS = 16
NW = NC * NS
CHUNK = 128
N_CHUNK = B // NW // CHUNK
N_IDX_ROWS = B // CHUNK


NBUF = 6
NK = 2 * N_CHUNK


def _sc_gather_body(uid_hbm, iid_hbm, utab_hbm, itab_hbm,
                    uout_hbm, iout_hbm, idx_v, rows_v, sem_g, sem_w):
    wid = lax.axis_index("s") * NC + lax.axis_index("c")
    r0 = wid * N_CHUNK

    pltpu.sync_copy(uid_hbm.at[pl.ds(r0, N_CHUNK)],
                    idx_v.at[pl.ds(0, N_CHUNK)])
    pltpu.sync_copy(iid_hbm.at[pl.ds(r0, N_CHUNK)],
                    idx_v.at[pl.ds(N_CHUNK, N_CHUNK)])

    srcs = [utab_hbm] * N_CHUNK + [itab_hbm] * N_CHUNK

    def dst(k):
        ref = uout_hbm if k < N_CHUNK else iout_hbm
        return ref.at[r0 + (k % N_CHUNK)]

    cps_g = [pltpu.async_copy(srcs[k].at[idx_v.at[k]], rows_v.at[k], sem_g)
             for k in range(2)]
    cps_w = [None] * NK
    for k in range(NK):
        nk = k + 2
        if nk < NK:
            if nk >= NBUF:
                cps_w[nk - NBUF].wait()
            cps_g.append(pltpu.async_copy(srcs[nk].at[idx_v.at[nk]],
                                          rows_v.at[nk % NBUF], sem_g))
        cps_g[k].wait()
        cps_w[k] = pltpu.async_copy(rows_v.at[k % NBUF], dst(k), sem_w)
    for k in range(NK - NBUF, NK):
        cps_w[k].wait()


def _sc_gather(user_ids2d, item_ids2d, user_table, item_table):
    mesh = plsc.VectorSubcoreMesh(core_axis_name="c", subcore_axis_name="s",
                                  num_cores=NC, num_subcores=NS)
    out_t = jax.ShapeDtypeStruct((N_IDX_ROWS, CHUNK, ED), jnp.float32)
    f = pl.kernel(
        _sc_gather_body,
        out_type=(out_t, out_t),
        mesh=mesh,
        scratch_types=[
            pltpu.VMEM((NK, CHUNK), jnp.int32),
            pltpu.VMEM((NBUF, CHUNK, ED), jnp.float32),
            pltpu.SemaphoreType.DMA,
            pltpu.SemaphoreType.DMA,
        ],
    )
    return f(user_ids2d, item_ids2d, user_table, item_table)


def _content_body(x_ref, w1_ref, b1_ref, w2_ref, b2_ref, c_ref):
    bf = jnp.bfloat16
    f32 = jnp.float32
    x = x_ref[...].astype(bf)
    h = jnp.maximum(
        jnp.dot(x, w1_ref[...].astype(bf),
                preferred_element_type=f32) + b1_ref[...], 0.0)
    c = jnp.dot(h.astype(bf), w2_ref[...].astype(bf),
                preferred_element_type=f32) + b2_ref[...]
    c_ref[...] = c.astype(bf)


def _content(x, W1, b1, W2, b2, bs=2048):
    nblk = B // bs
    row_blk = lambda idx: (idx, 0)
    whole = lambda idx: (0, 0)
    return pl.pallas_call(
        _content_body,
        grid=(nblk,),
        in_specs=[
            pl.BlockSpec((bs, NF), row_blk),
            pl.BlockSpec((NF, ED), whole),
            pl.BlockSpec((1, ED), whole),
            pl.BlockSpec((ED, ED), whole),
            pl.BlockSpec((1, ED), whole),
        ],
        out_specs=pl.BlockSpec((bs, ED), row_blk),
        out_shape=jax.ShapeDtypeStruct((B, ED), jnp.bfloat16),
    )(x, W1, b1.reshape(1, ED), W2, b2.reshape(1, ED))


def _combine_body(u_ref, i_ref, c_ref, w3_ref, b3_ref, w4_ref, b4_ref,
                  o_ref):
    bf = jnp.bfloat16
    f32 = jnp.float32
    acc = (jnp.dot(u_ref[...].astype(bf), w3_ref[0:ED, :].astype(bf),
                   preferred_element_type=f32)
           + jnp.dot(i_ref[...].astype(bf), w3_ref[ED:2 * ED, :].astype(bf),
                     preferred_element_type=f32)
           + jnp.dot(c_ref[...], w3_ref[2 * ED:3 * ED, :].astype(bf),
                     preferred_element_type=f32)
           + b3_ref[...])
    p = jnp.maximum(acc, 0.0)
    z = jnp.dot(p.astype(bf), w4_ref[...].astype(bf),
                preferred_element_type=f32) + b4_ref[...]
    s = jax.nn.sigmoid(z)
    o_ref[...] = s.reshape(o_ref.shape)


def _combine(u, i, c, W3, b3, W4, b4, bs=2048):
    nblk = B // bs
    row_blk = lambda idx: (idx, 0)
    whole = lambda idx: (0, 0)
    return pl.pallas_call(
        _combine_body,
        grid=(nblk,),
        in_specs=[
            pl.BlockSpec((bs, ED), row_blk),
            pl.BlockSpec((bs, ED), row_blk),
            pl.BlockSpec((bs, ED), row_blk),
            pl.BlockSpec((3 * ED, ED), whole),
            pl.BlockSpec((1, ED), whole),
            pl.BlockSpec((ED, 1), whole),
            pl.BlockSpec((1, 1), whole),
        ],
        out_specs=pl.BlockSpec((bs // 128, 128), row_blk),
        out_shape=jax.ShapeDtypeStruct((B // 128, 128), jnp.float32),
    )(u, i, c, W3, b3.reshape(1, ED), W4, b4.reshape(1, 1))


def kernel(user_ids, item_ids, item_features, user_table, item_table,
           W1, b1, W2, b2, W3, b3, W4, b4):
    uid2 = user_ids.astype(jnp.int32).reshape(N_IDX_ROWS, CHUNK)
    iid2 = item_ids.astype(jnp.int32).reshape(N_IDX_ROWS, CHUNK)
    u3, i3 = _sc_gather(uid2, iid2, user_table, item_table)
    c = _content(item_features, W1, b1, W2, b2)
    u = u3.reshape(B, ED)
    i = i3.reshape(B, ED)
    return _combine(u, i, c, W3, b3, W4, b4).reshape(B, 1)


# speedup vs baseline: 1.2341x; 1.0776x over previous
import functools

import jax
import jax.numpy as jnp
from jax import lax
from jax.experimental import pallas as pl
from jax.experimental.pallas import tpu as pltpu
from jax.experimental.pallas import tpu_sc as plsc

B = 16384
ED = 128
NF = 128

NC = 2
NS = 16
NW = NC * NS
CHUNK = 128
N_CHUNK = B // NW // CHUNK
N_IDX_ROWS = B // CHUNK


NBUF = 6
NK = 2 * N_CHUNK


def _sc_gather_body(uid_hbm, iid_hbm, utab_hbm, itab_hbm,
                    uout_hbm, iout_hbm, idx_v, rows_v, sem_g, sem_w):
    wid = lax.axis_index("s") * NC + lax.axis_index("c")
    r0 = wid * N_CHUNK

    pltpu.sync_copy(uid_hbm.at[pl.ds(r0, N_CHUNK)],
                    idx_v.at[pl.ds(0, N_CHUNK)])
    pltpu.sync_copy(iid_hbm.at[pl.ds(r0, N_CHUNK)],
                    idx_v.at[pl.ds(N_CHUNK, N_CHUNK)])

    srcs = [utab_hbm] * N_CHUNK + [itab_hbm] * N_CHUNK

    def dst(k):
        ref = uout_hbm if k < N_CHUNK else iout_hbm
        return ref.at[r0 + (k % N_CHUNK)]

    cps_g = [pltpu.async_copy(srcs[k].at[idx_v.at[k]], rows_v.at[k], sem_g)
             for k in range(2)]
    cps_w = [None] * NK
    for k in range(NK):
        nk = k + 2
        if nk < NK:
            if nk >= NBUF:
                cps_w[nk - NBUF].wait()
            cps_g.append(pltpu.async_copy(srcs[nk].at[idx_v.at[nk]],
                                          rows_v.at[nk % NBUF], sem_g))
        cps_g[k].wait()
        cps_w[k] = pltpu.async_copy(rows_v.at[k % NBUF], dst(k), sem_w)
    for k in range(NK - NBUF, NK):
        cps_w[k].wait()


def _sc_gather(user_ids2d, item_ids2d, user_table, item_table):
    mesh = plsc.VectorSubcoreMesh(core_axis_name="c", subcore_axis_name="s",
                                  num_cores=NC, num_subcores=NS)
    out_t = jax.ShapeDtypeStruct((N_IDX_ROWS, CHUNK, ED), jnp.float32)
    f = pl.kernel(
        _sc_gather_body,
        out_type=(out_t, out_t),
        mesh=mesh,
        scratch_types=[
            pltpu.VMEM((NK, CHUNK), jnp.int32),
            pltpu.VMEM((NBUF, CHUNK, ED), jnp.float32),
            pltpu.SemaphoreType.DMA,
            pltpu.SemaphoreType.DMA,
        ],
    )
    return f(user_ids2d, item_ids2d, user_table, item_table)


def _content_body(x_ref, w1_ref, b1_ref, w2_ref, b2_ref, c_ref):
    bf = jnp.bfloat16
    f32 = jnp.float32
    x = x_ref[...].astype(bf)
    h = jnp.maximum(
        jnp.dot(x, w1_ref[...].astype(bf),
                preferred_element_type=f32) + b1_ref[...], 0.0)
    c = jnp.dot(h.astype(bf), w2_ref[...].astype(bf),
                preferred_element_type=f32) + b2_ref[...]
    c_ref[...] = c.astype(bf)


def _content(x, W1, b1, W2, b2, bs=4096):
    nblk = B // bs
    row_blk = lambda idx: (idx, 0)
    whole = lambda idx: (0, 0)
    return pl.pallas_call(
        _content_body,
        grid=(nblk,),
        in_specs=[
            pl.BlockSpec((bs, NF), row_blk),
            pl.BlockSpec((NF, ED), whole),
            pl.BlockSpec((1, ED), whole),
            pl.BlockSpec((ED, ED), whole),
            pl.BlockSpec((1, ED), whole),
        ],
        out_specs=pl.BlockSpec((bs, ED), row_blk),
        out_shape=jax.ShapeDtypeStruct((B, ED), jnp.bfloat16),
    )(x, W1, b1.reshape(1, ED), W2, b2.reshape(1, ED))


def _combine_body(u_ref, i_ref, c_ref, w3_ref, b3_ref, w4_ref, b4_ref,
                  o_ref):
    bf = jnp.bfloat16
    f32 = jnp.float32
    acc = (jnp.dot(u_ref[...].astype(bf), w3_ref[0:ED, :].astype(bf),
                   preferred_element_type=f32)
           + jnp.dot(i_ref[...].astype(bf), w3_ref[ED:2 * ED, :].astype(bf),
                     preferred_element_type=f32)
           + jnp.dot(c_ref[...], w3_ref[2 * ED:3 * ED, :].astype(bf),
                     preferred_element_type=f32)
           + b3_ref[...])
    p = jnp.maximum(acc, 0.0)
    z = jnp.dot(p.astype(bf), w4_ref[...].astype(bf),
                preferred_element_type=f32) + b4_ref[...]
    s = jax.nn.sigmoid(z)
    o_ref[...] = s.reshape(o_ref.shape)


def _combine(u, i, c, W3, b3, W4, b4, bs=4096):
    nblk = B // bs
    row_blk = lambda idx: (idx, 0)
    whole = lambda idx: (0, 0)
    return pl.pallas_call(
        _combine_body,
        grid=(nblk,),
        in_specs=[
            pl.BlockSpec((bs, ED), row_blk),
            pl.BlockSpec((bs, ED), row_blk),
            pl.BlockSpec((bs, ED), row_blk),
            pl.BlockSpec((3 * ED, ED), whole),
            pl.BlockSpec((1, ED), whole),
            pl.BlockSpec((ED, 1), whole),
            pl.BlockSpec((1, 1), whole),
        ],
        out_specs=pl.BlockSpec((bs // 128, 128), row_blk),
        out_shape=jax.ShapeDtypeStruct((B // 128, 128), jnp.float32),
    )(u, i, c, W3, b3.reshape(1, ED), W4, b4.reshape(1, 1))


def kernel(user_ids, item_ids, item_features, user_table, item_table,
           W1, b1, W2, b2, W3, b3, W4, b4):
    uid2 = user_ids.astype(jnp.int32).reshape(N_IDX_ROWS, CHUNK)
    iid2 = item_ids.astype(jnp.int32).reshape(N_IDX_ROWS, CHUNK)
    u3, i3 = _sc_gather(uid2, iid2, user_table, item_table)
    c = _content(item_features, W1, b1, W2, b2)
    u = u3.reshape(B, ED)
    i = i3.reshape(B, ED)
    return _combine(u, i, c, W3, b3, W4, b4).reshape(B, 1)
